# baseline (speedup 1.0000x reference)
import functools

import jax
import jax.numpy as jnp
from jax import lax
from jax.experimental import pallas as pl
from jax.experimental.pallas import tpu as pltpu
from jax.experimental.pallas import tpu_sc as plsc

D = 128
CHUNK = 128
NC = 2
NS = 16
NW = NC * NS


@functools.lru_cache(maxsize=None)
def _make_gather(b: int, nbuf: int):
    n_full = b // CHUNK
    n_chunks = -(-b // CHUNK)
    n_slots = -(-n_chunks // NW)
    tail = b % CHUNK
    mesh = plsc.VectorSubcoreMesh(
        core_axis_name="c", subcore_axis_name="s",
        num_cores=NC, num_subcores=NS,
    )

    @functools.partial(
        pl.kernel,
        mesh=mesh,
        out_type=(jax.ShapeDtypeStruct((b, D), jnp.float32),
                  jax.ShapeDtypeStruct((CHUNK, D), jnp.float32)),
        scratch_types=[
            pltpu.VMEM((n_slots, CHUNK), jnp.int32)]
            + [pltpu.VMEM((CHUNK, D), jnp.float32)] * nbuf
            + [pltpu.SemaphoreType.DMA]
            + [pltpu.SemaphoreType.DMA] * (2 * nbuf),
    )
    def gather_kernel(table, idx, out, waste, idx_v, *bufs_sems):
        rows = bufs_sems[:nbuf]
        si = bufs_sems[nbuf]
        sg = bufs_sems[nbuf + 1:nbuf + 1 + nbuf]
        sw = bufs_sems[nbuf + 1 + nbuf:]
        wid = lax.axis_index("s") * NC + lax.axis_index("c")

        cis = [wid * n_slots + c for c in range(n_slots)]
        starts = [
            jnp.where(ci < n_full, ci * CHUNK,
                      jnp.where(ci == n_full, b - CHUNK,
                                (ci - n_chunks) * CHUNK)).astype(jnp.int32)
            for ci in cis
        ]

        idx_cps = []
        for c in range(n_slots):
            cp = pltpu.make_async_copy(
                idx.at[pl.ds(starts[c], CHUNK)], idx_v.at[c], si)
            cp.start()
            idx_cps.append(cp)
        for cp in idx_cps:
            cp.wait()

        def start_gather(c, bf):
            cp = pltpu.make_async_copy(table.at[idx_v.at[c]], rows[bf], sg[bf])
            cp.start()
            return cp

        def start_write(c, bf):
            ci = cis[c]

            @pl.when(ci < n_full)
            def _():
                pltpu.make_async_copy(
                    rows[bf], out.at[pl.ds(starts[c], CHUNK)], sw[bf]).start()

            if tail and (n_full - c) % n_slots == 0:
                @pl.when(ci == n_full)
                def _():
                    pltpu.make_async_copy(
                        rows[bf].at[pl.ds(CHUNK - tail, tail)],
                        out.at[pl.ds(b - tail, tail)], sw[bf]).start()
                    pltpu.make_async_copy(
                        rows[bf].at[pl.ds(0, CHUNK - tail)],
                        waste.at[pl.ds(0, CHUNK - tail)], sw[bf]).start()

            @pl.when(ci > n_full if tail else ci >= n_full)
            def _():
                pltpu.make_async_copy(
                    rows[bf], waste.at[pl.ds(0, CHUNK)], sw[bf]).start()

        def wait_write(bf):
            pltpu.make_async_copy(
                rows[bf], waste.at[pl.ds(0, CHUNK)], sw[bf]).wait()

        gathers = [None] * nbuf
        written = [False] * nbuf
        for c in range(nbuf - 1):
            gathers[c] = start_gather(c, c)
        for c in range(n_slots):
            bf = c % nbuf
            nxt = c + nbuf - 1
            if nxt < n_slots:
                bn = nxt % nbuf
                if written[bn]:
                    wait_write(bn)
                gathers[bn] = start_gather(nxt, bn)
            gathers[bf].wait()
            start_write(c, bf)
            written[bf] = True
        for c in range(max(0, n_slots - nbuf), n_slots):
            wait_write(c % nbuf)

    return gather_kernel


def kernel(graph, feat, select_idx):
    idx = select_idx.astype(jnp.int32)
    fn = _make_gather(idx.shape[0], 6)
    out, _ = fn(feat, idx)
    return out


# speedup vs baseline: 1.0521x; 1.0521x over previous
import functools

import jax
import jax.numpy as jnp
from jax import lax
from jax.experimental import pallas as pl
from jax.experimental.pallas import tpu as pltpu
from jax.experimental.pallas import tpu_sc as plsc

D = 128
CHUNK = 128
NC = 2
NS = 16
NW = NC * NS


@functools.lru_cache(maxsize=None)
def _make_gather(b: int, nbuf: int):
    n_full = b // CHUNK
    n_chunks = -(-b // CHUNK)
    n_slots = -(-n_chunks // NW)
    tail = b % CHUNK
    mesh = plsc.VectorSubcoreMesh(
        core_axis_name="c", subcore_axis_name="s",
        num_cores=NC, num_subcores=NS,
    )

    @functools.partial(
        pl.kernel,
        mesh=mesh,
        out_type=(jax.ShapeDtypeStruct((b, D), jnp.float32),
                  jax.ShapeDtypeStruct((CHUNK, D), jnp.float32)),
        scratch_types=[
            pltpu.VMEM((n_slots, CHUNK), jnp.int32)]
            + [pltpu.VMEM((CHUNK, D), jnp.float32)] * nbuf
            + [pltpu.SemaphoreType.DMA]
            + [pltpu.SemaphoreType.DMA] * (2 * nbuf),
    )
    def gather_kernel(table, idx, out, waste, idx_v, *bufs_sems):
        rows = bufs_sems[:nbuf]
        si = bufs_sems[nbuf]
        sg = bufs_sems[nbuf + 1:nbuf + 1 + nbuf]
        sw = bufs_sems[nbuf + 1 + nbuf:]
        wid = lax.axis_index("s") * NC + lax.axis_index("c")

        cis = [wid * n_slots + c for c in range(n_slots)]
        starts = [
            jnp.where(ci < n_full, ci * CHUNK,
                      jnp.where(ci == n_full, b - CHUNK,
                                (ci - n_chunks) * CHUNK)).astype(jnp.int32)
            for ci in cis
        ]

        idx_cps = []
        for c in range(n_slots):
            cp = pltpu.make_async_copy(
                idx.at[pl.ds(starts[c], CHUNK)], idx_v.at[c], si)
            cp.start()
            idx_cps.append(cp)
        for cp in idx_cps:
            cp.wait()

        def is_real(c):
            return cis[c] < n_chunks

        def start_gather(c, bf):
            @pl.when(is_real(c))
            def _():
                pltpu.make_async_copy(
                    table.at[idx_v.at[c]], rows[bf], sg[bf]).start()

        def wait_gather(c, bf):
            @pl.when(is_real(c))
            def _():
                pltpu.make_async_copy(
                    table.at[idx_v.at[c]], rows[bf], sg[bf]).wait()

        def start_write(c, bf):
            ci = cis[c]

            @pl.when(ci < n_full)
            def _():
                pltpu.make_async_copy(
                    rows[bf], out.at[pl.ds(starts[c], CHUNK)], sw[bf]).start()

            if tail and (n_full - c) % n_slots == 0:
                @pl.when(ci == n_full)
                def _():
                    pltpu.make_async_copy(
                        rows[bf].at[pl.ds(CHUNK - tail, tail)],
                        out.at[pl.ds(b - tail, tail)], sw[bf]).start()
                    pltpu.make_async_copy(
                        rows[bf].at[pl.ds(0, CHUNK - tail)],
                        waste.at[pl.ds(0, CHUNK - tail)], sw[bf]).start()

        def wait_write(c, bf):
            @pl.when(is_real(c))
            def _():
                pltpu.make_async_copy(
                    rows[bf], waste.at[pl.ds(0, CHUNK)], sw[bf]).wait()

        last_user = [None] * nbuf
        for c in range(nbuf - 1):
            start_gather(c, c)
        for c in range(n_slots):
            bf = c % nbuf
            nxt = c + nbuf - 1
            if nxt < n_slots:
                bn = nxt % nbuf
                if last_user[bn] is not None:
                    wait_write(last_user[bn], bn)
                start_gather(nxt, bn)
            wait_gather(c, bf)
            start_write(c, bf)
            last_user[bf] = c
        for c in range(max(0, n_slots - nbuf), n_slots):
            wait_write(c, c % nbuf)

    return gather_kernel


def kernel(graph, feat, select_idx):
    idx = select_idx.astype(jnp.int32)
    fn = _make_gather(idx.shape[0], 7)
    out, _ = fn(feat, idx)
    return out
